# baseline (speedup 1.0000x reference)
import functools

import jax
import jax.numpy as jnp
from jax import lax
from jax.experimental import pallas as pl
from jax.experimental.pallas import tpu as pltpu
from jax.experimental.pallas import tpu_sc as plsc

_LANES = 16
_NUM_CORES = 2
_NUM_SUBCORES = 16


def _build_sc_kernel(N, D, num_scales, rows_per_w, chunk_rows):
    n_chunks = rows_per_w // chunk_rows
    mesh = plsc.VectorSubcoreMesh(
        core_axis_name="c", subcore_axis_name="s",
        num_cores=_NUM_CORES, num_subcores=_NUM_SUBCORES,
    )
    num_cores = _NUM_CORES

    @functools.partial(
        pl.kernel,
        out_type=jax.ShapeDtypeStruct((N, D), jnp.float32),
        mesh=mesh,
        scratch_types=[
            pltpu.VMEM((num_scales, D), jnp.float32),
            pltpu.VMEM((rows_per_w,), jnp.int32),
            pltpu.VMEM((chunk_rows, D), jnp.float32),
        ],
    )
    def k(x_hbm, pos_hbm, tab_hbm, out_hbm, tab_v, idx_v, buf):
        wid = lax.axis_index("s") * num_cores + lax.axis_index("c")
        base = wid * rows_per_w
        pltpu.sync_copy(tab_hbm, tab_v)
        pltpu.sync_copy(pos_hbm.at[pl.ds(base, rows_per_w)], idx_v)

        def chunk_body(g, carry):
            row0 = base + g * chunk_rows
            pltpu.sync_copy(x_hbm.at[pl.ds(row0, chunk_rows)], buf)

            def group_body(t, c2):
                pos_vec = idx_v[pl.ds(g * chunk_rows + t * _LANES, _LANES)]
                for i in range(_LANES):
                    p = pos_vec[i]
                    r = t * _LANES + i

                    def col_body(j, c3, p=p, r=r):
                        sl = pl.ds(j * _LANES, _LANES)
                        plsc.addupdate(buf.at[r, sl], tab_v[p, sl])
                        return c3

                    lax.fori_loop(0, D // _LANES, col_body, 0, unroll=8)
                return c2

            lax.fori_loop(0, chunk_rows // _LANES, group_body, 0)
            pltpu.sync_copy(buf, out_hbm.at[pl.ds(row0, chunk_rows)])
            return carry

        lax.fori_loop(0, n_chunks, chunk_body, 0)

    return k


def kernel(inputs, num_scales, inputs_positions, scale_emb):
    B, S, D = inputs.shape
    N = B * S
    x = inputs.reshape(N, D)
    pos = inputs_positions.reshape(N)
    n_scales = scale_emb.shape[1]
    tab = scale_emb.reshape(n_scales, D)

    nw = _NUM_CORES * _NUM_SUBCORES
    rows_per_w = N // nw
    k = _build_sc_kernel(N, D, n_scales, rows_per_w, chunk_rows=64)
    return k(x, pos, tab).reshape(B, S, D)


# speedup vs baseline: 1.1361x; 1.1361x over previous
import functools

import jax
import jax.numpy as jnp
from jax import lax
from jax.experimental import pallas as pl
from jax.experimental.pallas import tpu as pltpu
from jax.experimental.pallas import tpu_sc as plsc

_LANES = 16
_NUM_CORES = 2
_NUM_SUBCORES = 16


def _build_sc_kernel(N, D, num_scales, rows_per_w, chunk_rows, nbuf, unroll):
    n_chunks = rows_per_w // chunk_rows
    n_rounds = n_chunks // nbuf
    assert n_chunks % nbuf == 0 and rows_per_w % chunk_rows == 0
    mesh = plsc.VectorSubcoreMesh(
        core_axis_name="c", subcore_axis_name="s",
        num_cores=_NUM_CORES, num_subcores=_NUM_SUBCORES,
    )

    scratch = [
        pltpu.VMEM((num_scales, D), jnp.float32),
        pltpu.VMEM((rows_per_w,), jnp.int32),
        pltpu.VMEM((nbuf * chunk_rows, D), jnp.float32),
        pltpu.SemaphoreType.DMA((nbuf,)),
        pltpu.SemaphoreType.DMA((nbuf,)),
    ]

    @functools.partial(
        pl.kernel,
        out_type=jax.ShapeDtypeStruct((N, D), jnp.float32),
        mesh=mesh,
        scratch_types=scratch,
    )
    def k(x_hbm, pos_hbm, tab_hbm, out_hbm, tab_v, idx_v, ring, in_sem, out_sem):
        wid = lax.axis_index("s") * _NUM_CORES + lax.axis_index("c")
        base = wid * rows_per_w
        pltpu.sync_copy(tab_hbm, tab_v)
        pltpu.sync_copy(pos_hbm.at[pl.ds(base, rows_per_w)], idx_v)

        def slot(b):
            return ring.at[pl.ds(b * chunk_rows, chunk_rows)]

        def start_in(g, b):
            pltpu.make_async_copy(
                x_hbm.at[pl.ds(base + g * chunk_rows, chunk_rows)],
                slot(b), in_sem.at[b],
            ).start()

        def wait_in(b):
            pltpu.make_async_copy(
                x_hbm.at[pl.ds(base, chunk_rows)], slot(b), in_sem.at[b],
            ).wait()

        def start_out(g, b):
            pltpu.make_async_copy(
                slot(b),
                out_hbm.at[pl.ds(base + g * chunk_rows, chunk_rows)],
                out_sem.at[b],
            ).start()

        def wait_out(b):
            pltpu.make_async_copy(
                slot(b), out_hbm.at[pl.ds(base, chunk_rows)], out_sem.at[b],
            ).wait()

        def compute(g, b):
            row_base = b * chunk_rows

            def group_body(t, c2):
                pos_vec = idx_v[pl.ds(g * chunk_rows + t * _LANES, _LANES)]
                r0 = row_base + t * _LANES
                for i in range(_LANES):
                    p = pos_vec[i]

                    def col_body(j, c3, p=p, r=r0 + i):
                        sl = pl.ds(j * _LANES, _LANES)
                        plsc.addupdate(ring.at[r, sl], tab_v[p, sl])
                        return c3

                    lax.fori_loop(0, D // _LANES, col_body, 0, unroll=unroll)
                return c2

            lax.fori_loop(0, chunk_rows // _LANES, group_body, 0)

        for b in range(nbuf):
            start_in(b, b)

        def slot_body(g, carry):
            b = lax.rem(g, nbuf)
            g2 = g + nbuf - 2
            b2 = lax.rem(g2, nbuf)

            @pl.when(jnp.logical_and(g >= 2, g2 < n_chunks))
            def _():
                wait_out(b2)
                start_in(g2, b2)

            wait_in(b)
            compute(g, b)
            start_out(g, b)
            return carry

        lax.fori_loop(0, n_chunks, slot_body, 0)

        for b in range(nbuf):
            wait_out(b)

    return k


def kernel(inputs, num_scales, inputs_positions, scale_emb):
    B, S, D = inputs.shape
    N = B * S
    x = inputs.reshape(N, D)
    pos = inputs_positions.reshape(N)
    n_scales = scale_emb.shape[1]
    tab = scale_emb.reshape(n_scales, D)

    nw = _NUM_CORES * _NUM_SUBCORES
    rows_per_w = N // nw
    k = _build_sc_kernel(N, D, n_scales, rows_per_w,
                         chunk_rows=16, nbuf=4, unroll=8)
    return k(x, pos, tab).reshape(B, S, D)


# speedup vs baseline: 1.7064x; 1.5020x over previous
import functools

import jax
import jax.numpy as jnp
from jax import lax
from jax.experimental import pallas as pl
from jax.experimental.pallas import tpu as pltpu
from jax.experimental.pallas import tpu_sc as plsc

_LANES = 16
_NUM_CORES = 2
_NUM_SUBCORES = 16


def _build_sc_kernel(N, D, num_scales, rows_per_w, chunk_rows, nbuf, unroll):
    n_chunks = rows_per_w // chunk_rows
    n_rounds = n_chunks // nbuf
    assert n_chunks % nbuf == 0 and rows_per_w % chunk_rows == 0
    mesh = plsc.VectorSubcoreMesh(
        core_axis_name="c", subcore_axis_name="s",
        num_cores=_NUM_CORES, num_subcores=_NUM_SUBCORES,
    )

    scratch = [
        pltpu.VMEM((num_scales, D), jnp.float32),
        pltpu.VMEM((rows_per_w,), jnp.int32),
        pltpu.VMEM((nbuf * chunk_rows, D), jnp.float32),
        pltpu.SemaphoreType.DMA((nbuf,)),
        pltpu.SemaphoreType.DMA((nbuf,)),
    ]

    @functools.partial(
        pl.kernel,
        out_type=jax.ShapeDtypeStruct((N, D), jnp.float32),
        mesh=mesh,
        scratch_types=scratch,
    )
    def k(x_hbm, pos_hbm, tab_hbm, out_hbm, tab_v, idx_v, ring, in_sem, out_sem):
        wid = lax.axis_index("s") * _NUM_CORES + lax.axis_index("c")
        base = wid * rows_per_w
        pltpu.sync_copy(tab_hbm, tab_v)
        pltpu.sync_copy(pos_hbm.at[pl.ds(base, rows_per_w)], idx_v)

        def slot(b):
            return ring.at[pl.ds(b * chunk_rows, chunk_rows)]

        def start_in(g, b):
            pltpu.make_async_copy(
                x_hbm.at[pl.ds(base + g * chunk_rows, chunk_rows)],
                slot(b), in_sem.at[b],
            ).start()

        def wait_in(b):
            pltpu.make_async_copy(
                x_hbm.at[pl.ds(base, chunk_rows)], slot(b), in_sem.at[b],
            ).wait()

        def start_out(g, b):
            pltpu.make_async_copy(
                slot(b),
                out_hbm.at[pl.ds(base + g * chunk_rows, chunk_rows)],
                out_sem.at[b],
            ).start()

        def wait_out(b):
            pltpu.make_async_copy(
                slot(b), out_hbm.at[pl.ds(base, chunk_rows)], out_sem.at[b],
            ).wait()

        def compute(g, b):
            row_base = b * chunk_rows

            def group_body(t, c2):
                pos_vec = idx_v[pl.ds(g * chunk_rows + t * _LANES, _LANES)]
                r0 = row_base + t * _LANES
                for i in range(_LANES):
                    p = pos_vec[i]

                    def col_body(jb, c3, p=p, r=r0 + i):
                        ts = [tab_v[p, pl.ds((jb * unroll + u) * _LANES, _LANES)]
                              for u in range(unroll)]
                        for u in range(unroll):
                            sl = pl.ds((jb * unroll + u) * _LANES, _LANES)
                            plsc.addupdate(ring.at[r, sl], ts[u])
                        return c3

                    lax.fori_loop(0, D // _LANES // unroll, col_body, 0)
                return c2

            lax.fori_loop(0, chunk_rows // _LANES, group_body, 0)

        for b in range(nbuf):
            start_in(b, b)

        def slot_body(g, carry):
            b = lax.rem(g, nbuf)
            g2 = g + nbuf - 2
            b2 = lax.rem(g2, nbuf)

            @pl.when(jnp.logical_and(g >= 2, g2 < n_chunks))
            def _():
                wait_out(b2)
                start_in(g2, b2)

            wait_in(b)
            compute(g, b)
            start_out(g, b)
            return carry

        lax.fori_loop(0, n_chunks, slot_body, 0)

        for b in range(nbuf):
            wait_out(b)

    return k


def kernel(inputs, num_scales, inputs_positions, scale_emb):
    B, S, D = inputs.shape
    N = B * S
    x = inputs.reshape(N, D)
    pos = inputs_positions.reshape(N)
    n_scales = scale_emb.shape[1]
    tab = scale_emb.reshape(n_scales, D)

    nw = _NUM_CORES * _NUM_SUBCORES
    rows_per_w = N // nw
    k = _build_sc_kernel(N, D, n_scales, rows_per_w,
                         chunk_rows=16, nbuf=4, unroll=8)
    return k(x, pos, tab).reshape(B, S, D)
